# baseline (speedup 1.0000x reference)
import functools

import jax
import jax.numpy as jnp
from jax import lax
from jax.experimental import pallas as pl
from jax.experimental.pallas import tpu as pltpu
from jax.experimental.pallas import tpu_sc as plsc

N = 10000
D = 128
E = 320000
NC = 2
NS = 16
BATCH = 128
K = 160
G = 16
NGRP = K // G
EP = NS * K * BATCH
N_PAD = 10240
ZROWS = N_PAD // NS


@functools.cache
def _make_sc_aggregate():
    mesh = plsc.VectorSubcoreMesh(
        core_axis_name="c", subcore_axis_name="s",
        num_cores=NC, num_subcores=NS)

    @functools.partial(
        pl.kernel,
        out_type=jax.ShapeDtypeStruct((NC, N_PAD, D), jnp.float32),
        mesh=mesh,
        scratch_types=[
            pltpu.VMEM((G, BATCH), jnp.int32),
            pltpu.VMEM((G, BATCH), jnp.int32),
            pltpu.VMEM((G, BATCH), jnp.int32),
            pltpu.VMEM((G, BATCH), jnp.int32),
            pltpu.VMEM((BATCH, D), jnp.float32),
            pltpu.VMEM((BATCH, D), jnp.float32),
            pltpu.VMEM_SHARED((N_PAD, D), jnp.float32),
            pltpu.SemaphoreType.DMA,
            pltpu.SemaphoreType.DMA,
            pltpu.SemaphoreType.DMA,
        ],
    )
    def sc_aggregate(pre_hbm, srcs_hbm, dsts_hbm, zeros_hbm, out_hbm,
                     src_a, dst_a, src_b, dst_b, bufa, bufb, agg_sh,
                     sema, semb, semi):
        c = lax.axis_index("c")
        s = lax.axis_index("s")
        pltpu.async_copy(srcs_hbm.at[c, s, pl.ds(0, G)], src_a, semi)
        pltpu.async_copy(dsts_hbm.at[c, s, pl.ds(0, G)], dst_a, semi)
        pltpu.sync_copy(zeros_hbm, agg_sh.at[pl.ds(s * ZROWS, ZROWS)])
        plsc.subcore_barrier()

        def do_group(g, src_v, dst_v, osrc, odst):
            pltpu.make_async_copy(
                srcs_hbm.at[c, s, pl.ds(g * G, G)], src_v, semi).wait()
            pltpu.make_async_copy(
                dsts_hbm.at[c, s, pl.ds(g * G, G)], dst_v, semi).wait()

            @pl.when(g + 1 < NGRP)
            def _prefetch_idx():
                pltpu.async_copy(
                    srcs_hbm.at[c, s, pl.ds((g + 1) * G, G)], osrc, semi)
                pltpu.async_copy(
                    dsts_hbm.at[c, s, pl.ds((g + 1) * G, G)], odst, semi)

            pltpu.async_copy(pre_hbm.at[src_v.at[0]], bufa, sema)

            def body(t, carry):
                j = 2 * t
                pltpu.make_async_copy(
                    pre_hbm.at[src_v.at[j]], bufa, sema).wait()
                pltpu.async_copy(pre_hbm.at[src_v.at[j + 1]], bufb, semb)
                pltpu.sync_copy(bufa, agg_sh.at[dst_v.at[j]], add=True)
                pltpu.make_async_copy(
                    pre_hbm.at[src_v.at[j + 1]], bufb, semb).wait()

                @pl.when(j + 2 < G)
                def _prefetch():
                    pltpu.async_copy(
                        pre_hbm.at[src_v.at[j + 2]], bufa, sema)

                pltpu.sync_copy(bufb, agg_sh.at[dst_v.at[j + 1]], add=True)
                return carry

            lax.fori_loop(0, G // 2, body, 0)

        def pair_body(g2, carry):
            do_group(2 * g2, src_a, dst_a, src_b, dst_b)
            do_group(2 * g2 + 1, src_b, dst_b, src_a, dst_a)
            return carry

        lax.fori_loop(0, NGRP // 2, pair_body, 0)
        plsc.subcore_barrier()
        pltpu.sync_copy(agg_sh.at[pl.ds(s * ZROWS, ZROWS)],
                        out_hbm.at[c, pl.ds(s * ZROWS, ZROWS)])

    return sc_aggregate


def _mm_body(s1_ref, s2_ref, w_ref, b_ref, o_ref):
    ci = pl.program_id(0)
    x = jnp.where(ci == 0, s1_ref[...], s2_ref[...])
    o_ref[0] = (
        jnp.dot(x, w_ref[...], preferred_element_type=jnp.float32)
        + b_ref[...]
    )


def _colsum_body(a_ref, o_ref):
    i = pl.program_id(0)

    @pl.when(i == 0)
    def _init():
        o_ref[...] = jnp.zeros_like(o_ref)

    h = a_ref[0]
    h = jnp.where(h > 0, h, 0.25 * h)
    o_ref[...] += jnp.sum(h, axis=0, keepdims=True)


def _score_body(a_ref, cs_ref, w_ref, b_ref, o_ref, wc_ref):
    i = pl.program_id(0)

    @pl.when(i == 0)
    def _init():
        cvec = jax.nn.sigmoid(cs_ref[...] * (1.0 / N))
        wc_ref[...] = lax.dot_general(
            cvec, w_ref[...], (((1,), (1,)), ((), ())),
            preferred_element_type=jnp.float32)

    h = a_ref[...]
    h = jnp.where(h > 0, h, 0.25 * h)
    o_ref[...] = jnp.sum(h * wc_ref[...], axis=1, keepdims=True) + b_ref[...]


def kernel(seq1, seq2, adj, W_gcn, b_gcn, W_disc, b_disc):
    pre = pl.pallas_call(
        _mm_body,
        grid=(2, 5),
        in_specs=[
            pl.BlockSpec((2000, D), lambda ci, i: (i, 0)),
            pl.BlockSpec((2000, D), lambda ci, i: (i, 0)),
            pl.BlockSpec((D, D), lambda ci, i: (0, 0)),
            pl.BlockSpec((1, D), lambda ci, i: (0, 0)),
        ],
        out_specs=pl.BlockSpec((1, 2000, D), lambda ci, i: (ci, i, 0)),
        out_shape=jax.ShapeDtypeStruct((NC, N, D), jnp.float32),
    )(seq1, seq2, W_gcn, b_gcn.reshape(1, D)).reshape(2 * N, D)

    src = adj[0]
    dst = adj[1]
    pad = EP - E
    srcp = jnp.concatenate([src, jnp.zeros((pad,), jnp.int32)])
    dstp = jnp.concatenate([dst, jnp.full((pad,), N, jnp.int32)])
    srcs = jnp.stack([srcp, srcp + N]).reshape(NC, NS, K, BATCH)
    dsts = jnp.stack([dstp, dstp]).reshape(NC, NS, K, BATCH)
    zeros = jnp.zeros((ZROWS, D), jnp.float32)

    agg = _make_sc_aggregate()(pre, srcs, dsts, zeros)

    colsum = pl.pallas_call(
        _colsum_body,
        grid=(5,),
        in_specs=[pl.BlockSpec((1, 2000, D), lambda i: (0, i, 0))],
        out_specs=pl.BlockSpec((1, D), lambda i: (0, 0)),
        out_shape=jax.ShapeDtypeStruct((1, D), jnp.float32),
    )(agg)

    scores = pl.pallas_call(
        _score_body,
        grid=(2 * N_PAD // 2048,),
        in_specs=[
            pl.BlockSpec((2048, D), lambda i: (i, 0)),
            pl.BlockSpec((1, D), lambda i: (0, 0)),
            pl.BlockSpec((D, D), lambda i: (0, 0)),
            pl.BlockSpec((1, 1), lambda i: (0, 0)),
        ],
        out_specs=pl.BlockSpec((2048, 1), lambda i: (i, 0)),
        out_shape=jax.ShapeDtypeStruct((2 * N_PAD, 1), jnp.float32),
        scratch_shapes=[pltpu.VMEM((1, D), jnp.float32)],
    )(agg.reshape(2 * N_PAD, D), colsum, W_disc, b_disc.reshape(1, 1))

    return scores.reshape(2, N_PAD)[:, :N].reshape(2 * N)


# speedup vs baseline: 1.0251x; 1.0251x over previous
import functools

import jax
import jax.numpy as jnp
from jax import lax
from jax.experimental import pallas as pl
from jax.experimental.pallas import tpu as pltpu
from jax.experimental.pallas import tpu_sc as plsc

N = 10000
D = 128
E = 320000
NC = 2
NS = 16
BATCH = 128
K = 160
G = 16
NGRP = K // G
EP = NS * K * BATCH
N_PAD = 10240
ZROWS = N_PAD // NS


@functools.cache
def _make_sc_aggregate():
    mesh = plsc.VectorSubcoreMesh(
        core_axis_name="c", subcore_axis_name="s",
        num_cores=NC, num_subcores=NS)

    @functools.partial(
        pl.kernel,
        out_type=jax.ShapeDtypeStruct((NC, N_PAD, D), jnp.float32),
        mesh=mesh,
        scratch_types=[
            pltpu.VMEM((G, BATCH), jnp.int32),
            pltpu.VMEM((G, BATCH), jnp.int32),
            pltpu.VMEM((G, BATCH), jnp.int32),
            pltpu.VMEM((G, BATCH), jnp.int32),
            pltpu.VMEM((BATCH, D), jnp.float32),
            pltpu.VMEM((BATCH, D), jnp.float32),
            pltpu.VMEM_SHARED((N_PAD, D), jnp.float32),
            pltpu.SemaphoreType.DMA,
            pltpu.SemaphoreType.DMA,
            pltpu.SemaphoreType.DMA,
        ],
    )
    def sc_aggregate(pre_hbm, srcs_hbm, dsts_hbm, out_hbm,
                     src_a, dst_a, src_b, dst_b, bufa, bufb, agg_sh,
                     sema, semb, semi):
        c = lax.axis_index("c")
        s = lax.axis_index("s")
        pltpu.async_copy(srcs_hbm.at[c, s, pl.ds(0, G)], src_a, semi)
        pltpu.async_copy(dsts_hbm.at[s, pl.ds(0, G)], dst_a, semi)
        zv = jnp.zeros((16,), jnp.float32)

        def zrow(r, carry):
            for kk in range(D // 16):
                bufa.at[r][pl.ds(16 * kk, 16)] = zv
            return carry

        lax.fori_loop(0, BATCH, zrow, 0)
        for z in range(ZROWS // BATCH):
            pltpu.sync_copy(
                bufa, agg_sh.at[pl.ds(s * ZROWS + z * BATCH, BATCH)])
        plsc.subcore_barrier()

        def do_group(g, src_v, dst_v, osrc, odst):
            pltpu.make_async_copy(
                srcs_hbm.at[c, s, pl.ds(g * G, G)], src_v, semi).wait()
            pltpu.make_async_copy(
                dsts_hbm.at[s, pl.ds(g * G, G)], dst_v, semi).wait()

            @pl.when(g + 1 < NGRP)
            def _prefetch_idx():
                pltpu.async_copy(
                    srcs_hbm.at[c, s, pl.ds((g + 1) * G, G)], osrc, semi)
                pltpu.async_copy(
                    dsts_hbm.at[s, pl.ds((g + 1) * G, G)], odst, semi)

            pltpu.async_copy(pre_hbm.at[src_v.at[0]], bufa, sema)

            def body(t, carry):
                j = 2 * t
                pltpu.make_async_copy(
                    pre_hbm.at[src_v.at[j]], bufa, sema).wait()
                pltpu.async_copy(pre_hbm.at[src_v.at[j + 1]], bufb, semb)
                pltpu.sync_copy(bufa, agg_sh.at[dst_v.at[j]], add=True)
                pltpu.make_async_copy(
                    pre_hbm.at[src_v.at[j + 1]], bufb, semb).wait()

                @pl.when(j + 2 < G)
                def _prefetch():
                    pltpu.async_copy(
                        pre_hbm.at[src_v.at[j + 2]], bufa, sema)

                pltpu.sync_copy(bufb, agg_sh.at[dst_v.at[j + 1]], add=True)
                return carry

            lax.fori_loop(0, G // 2, body, 0)

        def pair_body(g2, carry):
            do_group(2 * g2, src_a, dst_a, src_b, dst_b)
            do_group(2 * g2 + 1, src_b, dst_b, src_a, dst_a)
            return carry

        lax.fori_loop(0, NGRP // 2, pair_body, 0)
        plsc.subcore_barrier()
        pltpu.sync_copy(agg_sh.at[pl.ds(s * ZROWS, ZROWS)],
                        out_hbm.at[c, pl.ds(s * ZROWS, ZROWS)])

    return sc_aggregate


def _mm_body(s1_ref, s2_ref, w_ref, b_ref, o_ref):
    ci = pl.program_id(0)
    x = jnp.where(ci == 0, s1_ref[...], s2_ref[...])
    o_ref[0] = (
        jnp.dot(x, w_ref[...], preferred_element_type=jnp.float32)
        + b_ref[...]
    )


def _colsum_body(a_ref, o_ref):
    i = pl.program_id(0)

    @pl.when(i == 0)
    def _init():
        o_ref[...] = jnp.zeros_like(o_ref)

    h = a_ref[0]
    h = jnp.where(h > 0, h, 0.25 * h)
    o_ref[...] += jnp.sum(h, axis=0, keepdims=True)


def _score_body(a_ref, cs_ref, w_ref, b_ref, o_ref, wc_ref):
    i = pl.program_id(0)

    @pl.when(i == 0)
    def _init():
        cvec = jax.nn.sigmoid(cs_ref[...] * (1.0 / N))
        wc_ref[...] = lax.dot_general(
            cvec, w_ref[...], (((1,), (1,)), ((), ())),
            preferred_element_type=jnp.float32)

    h = a_ref[...]
    h = jnp.where(h > 0, h, 0.25 * h)
    o_ref[...] = jnp.sum(h * wc_ref[...], axis=1, keepdims=True) + b_ref[...]


def kernel(seq1, seq2, adj, W_gcn, b_gcn, W_disc, b_disc):
    pre = pl.pallas_call(
        _mm_body,
        grid=(2, 5),
        in_specs=[
            pl.BlockSpec((2000, D), lambda ci, i: (i, 0)),
            pl.BlockSpec((2000, D), lambda ci, i: (i, 0)),
            pl.BlockSpec((D, D), lambda ci, i: (0, 0)),
            pl.BlockSpec((1, D), lambda ci, i: (0, 0)),
        ],
        out_specs=pl.BlockSpec((1, 2000, D), lambda ci, i: (ci, i, 0)),
        out_shape=jax.ShapeDtypeStruct((NC, N, D), jnp.float32),
    )(seq1, seq2, W_gcn, b_gcn.reshape(1, D)).reshape(2 * N, D)

    src = adj[0]
    dst = adj[1]
    pad = EP - E
    srcp = jnp.concatenate([src, jnp.zeros((pad,), jnp.int32)])
    dstp = jnp.concatenate([dst, jnp.full((pad,), N, jnp.int32)])
    srcs = jnp.stack([srcp, srcp + N]).reshape(NC, NS, K, BATCH)
    dsts = dstp.reshape(NS, K, BATCH)
    agg = _make_sc_aggregate()(pre, srcs, dsts)

    colsum = pl.pallas_call(
        _colsum_body,
        grid=(5,),
        in_specs=[pl.BlockSpec((1, 2000, D), lambda i: (0, i, 0))],
        out_specs=pl.BlockSpec((1, D), lambda i: (0, 0)),
        out_shape=jax.ShapeDtypeStruct((1, D), jnp.float32),
    )(agg)

    scores = pl.pallas_call(
        _score_body,
        grid=(2 * N_PAD // 2048,),
        in_specs=[
            pl.BlockSpec((2048, D), lambda i: (i, 0)),
            pl.BlockSpec((1, D), lambda i: (0, 0)),
            pl.BlockSpec((D, D), lambda i: (0, 0)),
            pl.BlockSpec((1, 1), lambda i: (0, 0)),
        ],
        out_specs=pl.BlockSpec((2048, 1), lambda i: (i, 0)),
        out_shape=jax.ShapeDtypeStruct((2 * N_PAD, 1), jnp.float32),
        scratch_shapes=[pltpu.VMEM((1, D), jnp.float32)],
    )(agg.reshape(2 * N_PAD, D), colsum, W_disc, b_disc.reshape(1, 1))

    return scores.reshape(2, N_PAD)[:, :N].reshape(2 * N)


# speedup vs baseline: 1.1026x; 1.0756x over previous
import functools

import jax
import jax.numpy as jnp
from jax import lax
from jax.experimental import pallas as pl
from jax.experimental.pallas import tpu as pltpu
from jax.experimental.pallas import tpu_sc as plsc

N = 10000
D = 128
E = 320000
NC = 2
NS = 16
BATCH = 128
K = 160
G = 16
NGRP = K // G
EP = NS * K * BATCH
N_PAD = 10240
ZROWS = N_PAD // NS


@functools.cache
def _make_sc_aggregate():
    mesh = plsc.VectorSubcoreMesh(
        core_axis_name="c", subcore_axis_name="s",
        num_cores=NC, num_subcores=NS)

    @functools.partial(
        pl.kernel,
        out_type=jax.ShapeDtypeStruct((NC, N_PAD, D), jnp.float32),
        mesh=mesh,
        scratch_types=[
            pltpu.VMEM((G, BATCH), jnp.int32),
            pltpu.VMEM((G, BATCH), jnp.int32),
            pltpu.VMEM((G, BATCH), jnp.int32),
            pltpu.VMEM((G, BATCH), jnp.int32),
            pltpu.VMEM((BATCH, D), jnp.float32),
            pltpu.VMEM((BATCH, D), jnp.float32),
            pltpu.VMEM_SHARED((N_PAD, D), jnp.float32),
            pltpu.SemaphoreType.DMA,
            pltpu.SemaphoreType.DMA,
            pltpu.SemaphoreType.DMA,
        ],
    )
    def sc_aggregate(pre_hbm, srcs_hbm, dsts_hbm, out_hbm,
                     src_a, dst_a, src_b, dst_b, bufa, bufb, agg_sh,
                     sema, semb, semi):
        c = lax.axis_index("c")
        s = lax.axis_index("s")
        pltpu.async_copy(srcs_hbm.at[c, s, pl.ds(0, G)], src_a, semi)
        pltpu.async_copy(dsts_hbm.at[s, pl.ds(0, G)], dst_a, semi)
        zv = jnp.zeros((16,), jnp.float32)

        def zrow(r, carry):
            for kk in range(D // 16):
                bufa.at[r][pl.ds(16 * kk, 16)] = zv
            return carry

        lax.fori_loop(0, BATCH, zrow, 0)
        for z in range(ZROWS // BATCH):
            pltpu.sync_copy(
                bufa, agg_sh.at[pl.ds(s * ZROWS + z * BATCH, BATCH)])
        plsc.subcore_barrier()

        def do_group(g, src_v, dst_v, osrc, odst):
            pltpu.make_async_copy(
                srcs_hbm.at[c, s, pl.ds(g * G, G)], src_v, semi).wait()
            pltpu.make_async_copy(
                dsts_hbm.at[s, pl.ds(g * G, G)], dst_v, semi).wait()

            @pl.when(g + 1 < NGRP)
            def _prefetch_idx():
                pltpu.async_copy(
                    srcs_hbm.at[c, s, pl.ds((g + 1) * G, G)], osrc, semi)
                pltpu.async_copy(
                    dsts_hbm.at[s, pl.ds((g + 1) * G, G)], odst, semi)

            pltpu.async_copy(pre_hbm.at[src_v.at[0]], bufa, sema)

            def body(t, carry):
                j = 2 * t
                pltpu.make_async_copy(
                    pre_hbm.at[src_v.at[j]], bufa, sema).wait()
                pltpu.async_copy(pre_hbm.at[src_v.at[j + 1]], bufb, semb)
                pltpu.sync_copy(bufa, agg_sh.at[dst_v.at[j]], add=True)
                pltpu.make_async_copy(
                    pre_hbm.at[src_v.at[j + 1]], bufb, semb).wait()

                @pl.when(j + 2 < G)
                def _prefetch():
                    pltpu.async_copy(
                        pre_hbm.at[src_v.at[j + 2]], bufa, sema)

                pltpu.sync_copy(bufb, agg_sh.at[dst_v.at[j + 1]], add=True)
                return carry

            lax.fori_loop(0, G // 2, body, 0)

        def pair_body(g2, carry):
            do_group(2 * g2, src_a, dst_a, src_b, dst_b)
            do_group(2 * g2 + 1, src_b, dst_b, src_a, dst_a)
            return carry

        lax.fori_loop(0, NGRP // 2, pair_body, 0)
        plsc.subcore_barrier()
        pltpu.sync_copy(agg_sh.at[pl.ds(s * ZROWS, ZROWS)],
                        out_hbm.at[c, pl.ds(s * ZROWS, ZROWS)])

    return sc_aggregate


def _mm_body(s1_ref, s2_ref, w_ref, b_ref, o_ref):
    ci = pl.program_id(0)
    x = jnp.where(ci == 0, s1_ref[...], s2_ref[...])
    o_ref[0] = (
        jnp.dot(x, w_ref[...], preferred_element_type=jnp.float32)
        + b_ref[...]
    )


def _colsum_body(a_ref, o_ref):
    i = pl.program_id(0)

    @pl.when(i == 0)
    def _init():
        o_ref[...] = jnp.zeros_like(o_ref)

    h = a_ref[0]
    h = jnp.where(h > 0, h, 0.25 * h)
    o_ref[...] += jnp.sum(h, axis=0, keepdims=True)


def _score_body(a_ref, cs_ref, w_ref, b_ref, o_ref, wc_ref):
    i = pl.program_id(0)

    @pl.when(i == 0)
    def _init():
        cvec = jax.nn.sigmoid(cs_ref[...] * (1.0 / N))
        wc_ref[...] = lax.dot_general(
            cvec, w_ref[...], (((1,), (1,)), ((), ())),
            preferred_element_type=jnp.float32)

    h = a_ref[...]
    h = jnp.where(h > 0, h, 0.25 * h)
    o_ref[...] = jnp.sum(h * wc_ref[...], axis=1, keepdims=True) + b_ref[...]


def kernel(seq1, seq2, adj, W_gcn, b_gcn, W_disc, b_disc):
    pre = pl.pallas_call(
        _mm_body,
        grid=(2, 5),
        in_specs=[
            pl.BlockSpec((2000, D), lambda ci, i: (i, 0)),
            pl.BlockSpec((2000, D), lambda ci, i: (i, 0)),
            pl.BlockSpec((D, D), lambda ci, i: (0, 0)),
            pl.BlockSpec((1, D), lambda ci, i: (0, 0)),
        ],
        out_specs=pl.BlockSpec((1, 2000, D), lambda ci, i: (ci, i, 0)),
        out_shape=jax.ShapeDtypeStruct((NC, N, D), jnp.float32),
    )(seq1, seq2, W_gcn, b_gcn.reshape(1, D))
    pre = jnp.transpose(pre, (1, 0, 2)).reshape(2 * N, D)

    src = adj[0]
    dst = adj[1]
    pad = EP - E
    srcp = jnp.concatenate([src, jnp.zeros((pad,), jnp.int32)])
    dstp = jnp.concatenate([dst, jnp.full((pad,), N, jnp.int32)])
    srcs = jnp.stack([2 * srcp, 2 * srcp + 1]).reshape(NC, NS, K, BATCH)
    dsts = dstp.reshape(NS, K, BATCH)
    agg = _make_sc_aggregate()(pre, srcs, dsts)

    colsum = pl.pallas_call(
        _colsum_body,
        grid=(5,),
        in_specs=[pl.BlockSpec((1, 2000, D), lambda i: (0, i, 0))],
        out_specs=pl.BlockSpec((1, D), lambda i: (0, 0)),
        out_shape=jax.ShapeDtypeStruct((1, D), jnp.float32),
    )(agg)

    scores = pl.pallas_call(
        _score_body,
        grid=(2 * N_PAD // 2048,),
        in_specs=[
            pl.BlockSpec((2048, D), lambda i: (i, 0)),
            pl.BlockSpec((1, D), lambda i: (0, 0)),
            pl.BlockSpec((D, D), lambda i: (0, 0)),
            pl.BlockSpec((1, 1), lambda i: (0, 0)),
        ],
        out_specs=pl.BlockSpec((2048, 1), lambda i: (i, 0)),
        out_shape=jax.ShapeDtypeStruct((2 * N_PAD, 1), jnp.float32),
        scratch_shapes=[pltpu.VMEM((1, D), jnp.float32)],
    )(agg.reshape(2 * N_PAD, D), colsum, W_disc, b_disc.reshape(1, 1))

    return scores.reshape(2, N_PAD)[:, :N].reshape(2 * N)


# speedup vs baseline: 1.1838x; 1.0737x over previous
import functools

import jax
import jax.numpy as jnp
from jax import lax
from jax.experimental import pallas as pl
from jax.experimental.pallas import tpu as pltpu
from jax.experimental.pallas import tpu_sc as plsc

N = 10000
D = 128
E = 320000
NC = 2
NS = 16
BATCH = 128
K = 160
G = 16
NGRP = K // G
EP = NS * K * BATCH
N_PAD = 10240
ZROWS = N_PAD // NS


@functools.cache
def _make_sc_aggregate():
    mesh = plsc.VectorSubcoreMesh(
        core_axis_name="c", subcore_axis_name="s",
        num_cores=NC, num_subcores=NS)

    @functools.partial(
        pl.kernel,
        out_type=jax.ShapeDtypeStruct((NC, N_PAD, D), jnp.float32),
        mesh=mesh,
        scratch_types=[
            pltpu.VMEM((G, BATCH), jnp.int32),
            pltpu.VMEM((G, BATCH), jnp.int32),
            pltpu.VMEM((G, BATCH), jnp.int32),
            pltpu.VMEM((G, BATCH), jnp.int32),
            pltpu.VMEM((BATCH, D), jnp.float32),
            pltpu.VMEM((BATCH, D), jnp.float32),
            pltpu.VMEM_SHARED((N_PAD, D), jnp.float32),
            pltpu.SemaphoreType.DMA,
            pltpu.SemaphoreType.DMA,
            pltpu.SemaphoreType.DMA,
        ],
    )
    def sc_aggregate(pre_hbm, srcs_hbm, dsts_hbm, out_hbm,
                     src_a, dst_a, src_b, dst_b, bufa, bufb, agg_sh,
                     sema, semb, semi):
        c = lax.axis_index("c")
        s = lax.axis_index("s")
        pltpu.async_copy(srcs_hbm.at[c, s, pl.ds(0, G)], src_a, semi)
        pltpu.async_copy(dsts_hbm.at[s, pl.ds(0, G)], dst_a, semi)
        zv = jnp.zeros((16,), jnp.float32)

        def zrow(r, carry):
            for kk in range(D // 16):
                bufa.at[r][pl.ds(16 * kk, 16)] = zv
            return carry

        lax.fori_loop(0, BATCH, zrow, 0)
        for z in range(ZROWS // BATCH):
            pltpu.sync_copy(
                bufa, agg_sh.at[pl.ds(s * ZROWS + z * BATCH, BATCH)])
        plsc.subcore_barrier()

        def idx_wait(g, src_v, dst_v):
            pltpu.make_async_copy(
                srcs_hbm.at[c, s, pl.ds(g * G, G)], src_v, semi).wait()
            pltpu.make_async_copy(
                dsts_hbm.at[s, pl.ds(g * G, G)], dst_v, semi).wait()

        def idx_fetch(g, src_v, dst_v):
            pltpu.async_copy(
                srcs_hbm.at[c, s, pl.ds(g * G, G)], src_v, semi)
            pltpu.async_copy(
                dsts_hbm.at[s, pl.ds(g * G, G)], dst_v, semi)

        idx_wait(0, src_a, dst_a)
        idx_fetch(1, src_b, dst_b)
        pltpu.async_copy(pre_hbm.at[src_a.at[0]], bufa, sema)
        pltpu.async_copy(pre_hbm.at[src_a.at[1]], bufb, semb)

        def do_group(g, src_v, dst_v, osrc, odst):
            def body(t, carry):
                j = 2 * t
                pltpu.make_async_copy(
                    pre_hbm.at[src_v.at[j]], bufa, sema).wait()
                pltpu.sync_copy(bufa, agg_sh.at[dst_v.at[j]], add=True)

                @pl.when(j + 2 < G)
                def _next_a():
                    pltpu.async_copy(
                        pre_hbm.at[src_v.at[j + 2]], bufa, sema)

                @pl.when((j + 2 >= G) & (g + 1 < NGRP))
                def _wrap_a():
                    idx_wait(g + 1, osrc, odst)
                    pltpu.async_copy(pre_hbm.at[osrc.at[0]], bufa, sema)

                pltpu.make_async_copy(
                    pre_hbm.at[src_v.at[j + 1]], bufb, semb).wait()
                pltpu.sync_copy(bufb, agg_sh.at[dst_v.at[j + 1]], add=True)

                @pl.when(j + 3 < G)
                def _next_b():
                    pltpu.async_copy(
                        pre_hbm.at[src_v.at[j + 3]], bufb, semb)

                @pl.when((j + 3 >= G) & (g + 1 < NGRP))
                def _wrap_b():
                    pltpu.async_copy(pre_hbm.at[osrc.at[1]], bufb, semb)

                return carry

            lax.fori_loop(0, G // 2, body, 0)

            @pl.when(g + 2 < NGRP)
            def _fetch_next():
                idx_fetch(g + 2, src_v, dst_v)

        def pair_body(g2, carry):
            do_group(2 * g2, src_a, dst_a, src_b, dst_b)
            do_group(2 * g2 + 1, src_b, dst_b, src_a, dst_a)
            return carry

        lax.fori_loop(0, NGRP // 2, pair_body, 0)
        plsc.subcore_barrier()
        pltpu.sync_copy(agg_sh.at[pl.ds(s * ZROWS, ZROWS)],
                        out_hbm.at[c, pl.ds(s * ZROWS, ZROWS)])

    return sc_aggregate


def _mm_body(s1_ref, s2_ref, w_ref, b_ref, o_ref):
    ci = pl.program_id(0)
    x = jnp.where(ci == 0, s1_ref[...], s2_ref[...])
    o_ref[0] = (
        jnp.dot(x, w_ref[...], preferred_element_type=jnp.float32)
        + b_ref[...]
    )


def _colsum_body(a_ref, o_ref):
    i = pl.program_id(0)

    @pl.when(i == 0)
    def _init():
        o_ref[...] = jnp.zeros_like(o_ref)

    h = a_ref[0]
    h = jnp.where(h > 0, h, 0.25 * h)
    o_ref[...] += jnp.sum(h, axis=0, keepdims=True)


def _score_body(a_ref, cs_ref, w_ref, b_ref, o_ref, wc_ref):
    i = pl.program_id(0)

    @pl.when(i == 0)
    def _init():
        cvec = jax.nn.sigmoid(cs_ref[...] * (1.0 / N))
        wc_ref[...] = lax.dot_general(
            cvec, w_ref[...], (((1,), (1,)), ((), ())),
            preferred_element_type=jnp.float32)

    h = a_ref[...]
    h = jnp.where(h > 0, h, 0.25 * h)
    o_ref[...] = jnp.sum(h * wc_ref[...], axis=1, keepdims=True) + b_ref[...]


def kernel(seq1, seq2, adj, W_gcn, b_gcn, W_disc, b_disc):
    pre = pl.pallas_call(
        _mm_body,
        grid=(2, 5),
        in_specs=[
            pl.BlockSpec((2000, D), lambda ci, i: (i, 0)),
            pl.BlockSpec((2000, D), lambda ci, i: (i, 0)),
            pl.BlockSpec((D, D), lambda ci, i: (0, 0)),
            pl.BlockSpec((1, D), lambda ci, i: (0, 0)),
        ],
        out_specs=pl.BlockSpec((1, 2000, D), lambda ci, i: (ci, i, 0)),
        out_shape=jax.ShapeDtypeStruct((NC, N, D), jnp.float32),
    )(seq1, seq2, W_gcn, b_gcn.reshape(1, D))
    pre = jnp.transpose(pre, (1, 0, 2)).reshape(2 * N, D)

    src = adj[0]
    dst = adj[1]
    pad = EP - E
    srcp = jnp.concatenate([src, jnp.zeros((pad,), jnp.int32)])
    dstp = jnp.concatenate([dst, jnp.full((pad,), N, jnp.int32)])
    srcs = jnp.stack([2 * srcp, 2 * srcp + 1]).reshape(NC, NS, K, BATCH)
    dsts = dstp.reshape(NS, K, BATCH)
    agg = _make_sc_aggregate()(pre, srcs, dsts)

    colsum = pl.pallas_call(
        _colsum_body,
        grid=(5,),
        in_specs=[pl.BlockSpec((1, 2000, D), lambda i: (0, i, 0))],
        out_specs=pl.BlockSpec((1, D), lambda i: (0, 0)),
        out_shape=jax.ShapeDtypeStruct((1, D), jnp.float32),
    )(agg)

    scores = pl.pallas_call(
        _score_body,
        grid=(2 * N_PAD // 2048,),
        in_specs=[
            pl.BlockSpec((2048, D), lambda i: (i, 0)),
            pl.BlockSpec((1, D), lambda i: (0, 0)),
            pl.BlockSpec((D, D), lambda i: (0, 0)),
            pl.BlockSpec((1, 1), lambda i: (0, 0)),
        ],
        out_specs=pl.BlockSpec((2048, 1), lambda i: (i, 0)),
        out_shape=jax.ShapeDtypeStruct((2 * N_PAD, 1), jnp.float32),
        scratch_shapes=[pltpu.VMEM((1, D), jnp.float32)],
    )(agg.reshape(2 * N_PAD, D), colsum, W_disc, b_disc.reshape(1, 1))

    return scores.reshape(2, N_PAD)[:, :N].reshape(2 * N)
